# baseline (speedup 1.0000x reference)
import functools

import jax
import jax.numpy as jnp
from jax import lax
from jax.experimental import pallas as pl
from jax.experimental.pallas import tpu as pltpu
from jax.experimental.pallas import tpu_sc as plsc

A, B, C0, D = 16, 16, 16, 128
N = A * B * C0 * 8 * D
NW = 32
CELLS_PER_W = N // NW
CHUNK = 2048
NBUF = 4
ITERS = CELLS_PER_W // CHUNK
T_PER_CHUNK = (CHUNK // 1024) * 8


def _body(input_hbm, mean_hbm, sig_hbm, eps_hbm, out_hbm, outz_hbm, *rest):
    bufs = tuple(tuple(rest[b * 6:(b + 1) * 6]) for b in range(NBUF))
    zero_v, sem_in, sem_out = rest[NBUF * 6:]
    wid = lax.axis_index("s") * 2 + lax.axis_index("c")
    iota = lax.iota(jnp.int32, 16)

    def base_of(it):
        return pl.multiple_of(wid * CELLS_PER_W + it * CHUNK, CHUNK)

    def in_copies(it, b):
        base = base_of(it)
        iv, mv, sv, ev, _, _ = bufs[b]
        return [
            pltpu.make_async_copy(
                input_hbm.at[pl.ds(pl.multiple_of(base // 8, CHUNK // 8),
                                   CHUNK // 8)],
                iv, sem_in.at[b]),
            pltpu.make_async_copy(
                mean_hbm.at[pl.ds(pl.multiple_of(base * 4, CHUNK * 4),
                                  CHUNK * 4)],
                mv, sem_in.at[b]),
            pltpu.make_async_copy(
                sig_hbm.at[pl.ds(pl.multiple_of(base * 4, CHUNK * 4),
                                 CHUNK * 4)],
                sv, sem_in.at[b]),
            pltpu.make_async_copy(
                eps_hbm.at[pl.ds(base, CHUNK)], ev, sem_in.at[b]),
        ]

    def out_copies(it, b):
        base = base_of(it)
        ov_p, ov_n = bufs[b][4], bufs[b][5]
        return [
            pltpu.make_async_copy(
                ov_p, out_hbm.at[pl.ds(base, CHUNK)], sem_out.at[b]),
            pltpu.make_async_copy(
                ov_n, out_hbm.at[pl.ds(N + base, CHUNK)], sem_out.at[b]),
            pltpu.make_async_copy(
                zero_v, outz_hbm.at[pl.ds(base, CHUNK)], sem_out.at[b]),
            pltpu.make_async_copy(
                zero_v, outz_hbm.at[pl.ds(N + base, CHUNK)], sem_out.at[b]),
        ]

    def compute(b):
        iv, mv, sv, ev, ov_p, ov_n = bufs[b]

        def _row_body(row, _):
            x0 = row * 128
            c0 = row * 1024
            b0 = row * 4096
            for j in range(8):
                d0 = j * 16
                x = iv[pl.ds(x0 + d0, 16)]
                xp = jnp.maximum(x, 0.0).astype(jnp.int32)
                xn = jnp.maximum(-x, 0.0).astype(jnp.int32)
                vecj = (b0 + d0) + iota
                for k in range(8):
                    sh = 7 - 2 * k
                    if sh >= 0:
                        cp = (xp >> sh) & 0x180
                        cn = (xn >> sh) & 0x180
                    else:
                        cp = (xp << -sh) & 0x180
                        cn = (xn << -sh) & 0x180
                    veck = vecj + k * 512
                    mp = plsc.load_gather(mv, [veck + cp])
                    sp = plsc.load_gather(sv, [veck + cp])
                    mn = plsc.load_gather(mv, [veck + cn])
                    sn = plsc.load_gather(sv, [veck + cn])
                    off = c0 + d0 + k * 128
                    e = ev[pl.ds(off, 16)]
                    ov_p[pl.ds(off, 16)] = mp + e * sp
                    ov_n[pl.ds(off, 16)] = mn + e * sn
            return 0

        lax.fori_loop(0, CHUNK // 1024, _row_body, 0)

    zvec = jnp.zeros((16,), jnp.float32)

    def zinit(g, _):
        zero_v[pl.ds(g * 16, 16)] = zvec
        return 0

    lax.fori_loop(0, CHUNK // 16, zinit, 0)

    for b in range(NBUF):
        for c in in_copies(b, b):
            c.start()

    def loop_ring(i, _):
        for b in range(NBUF):
            it = i * NBUF + b
            for c in in_copies(it, b):
                c.wait()

            @pl.when(it >= NBUF)
            def _():
                for c in out_copies(it - NBUF, b):
                    c.wait()

            compute(b)
            for c in out_copies(it, b):
                c.start()

            @pl.when(it + NBUF < ITERS)
            def _():
                for c in in_copies(it + NBUF, b):
                    c.start()
        return 0

    lax.fori_loop(0, ITERS // NBUF, loop_ring, 0)

    for b in range(NBUF):
        for c in out_copies(ITERS - NBUF + b, b):
            c.wait()


@jax.jit
def kernel(input, mean_G, sig_G, eps):
    mesh = plsc.VectorSubcoreMesh(core_axis_name="c", subcore_axis_name="s",
                                  num_cores=2, num_subcores=16)
    run = functools.partial(
        pl.kernel,
        out_type=(jax.ShapeDtypeStruct((2 * N,), jnp.float32),
                  jax.ShapeDtypeStruct((2 * N,), jnp.float32)),
        mesh=mesh,
        compiler_params=pltpu.CompilerParams(needs_layout_passes=False),
        scratch_types=(
            [pltpu.VMEM((CHUNK // 8,), jnp.float32),
             pltpu.VMEM((CHUNK * 4,), jnp.float32),
             pltpu.VMEM((CHUNK * 4,), jnp.float32),
             pltpu.VMEM((CHUNK,), jnp.float32),
             pltpu.VMEM((CHUNK,), jnp.float32),
             pltpu.VMEM((CHUNK,), jnp.float32)] * NBUF
            + [pltpu.VMEM((CHUNK,), jnp.float32),
               pltpu.SemaphoreType.DMA((NBUF,)),
               pltpu.SemaphoreType.DMA((NBUF,))]),
    )(_body)
    mean_lin = jnp.transpose(mean_G, (0, 1, 2, 4, 3)).reshape(-1)
    sig_lin = jnp.transpose(sig_G, (0, 1, 2, 4, 3)).reshape(-1)
    out, outz = run(input.reshape(-1), mean_lin, sig_lin, eps.reshape(-1))
    g_inactive = out.reshape(2, A, B, C0 * 8, D)
    g_active = outz.reshape(2, A, B, C0 * 8, D)
    return (g_active, g_inactive)


# speedup vs baseline: 1.5376x; 1.5376x over previous
import functools

import jax
import jax.numpy as jnp
from jax import lax
from jax.experimental import pallas as pl
from jax.experimental.pallas import tpu as pltpu
from jax.experimental.pallas import tpu_sc as plsc

A, B, C0, D = 16, 16, 16, 128
N = A * B * C0 * 8 * D
NW = 32
CELLS_PER_W = N // NW
CHUNK = 2048
NBUF = 4
ITERS = CELLS_PER_W // CHUNK
T_PER_CHUNK = (CHUNK // 1024) * 8


def _body(input_hbm, mean_hbm, sig_hbm, eps_hbm, out_hbm, outz_hbm, *rest):
    bufs = tuple(tuple(rest[b * 6:(b + 1) * 6]) for b in range(NBUF))
    zero_v, sem_in, sem_out = rest[NBUF * 6:]
    wid = lax.axis_index("s") * 2 + lax.axis_index("c")
    iota = lax.iota(jnp.int32, 16)

    def base_of(it):
        return pl.multiple_of(wid * CELLS_PER_W + it * CHUNK, CHUNK)

    def in_copies(it, b):
        base = base_of(it)
        iv, mv, sv, ev, _, _ = bufs[b]
        return [
            pltpu.make_async_copy(
                input_hbm.at[pl.ds(pl.multiple_of(base // 8, CHUNK // 8),
                                   CHUNK // 8)],
                iv, sem_in.at[b]),
            pltpu.make_async_copy(
                mean_hbm.at[pl.ds(pl.multiple_of(base * 4, CHUNK * 4),
                                  CHUNK * 4)],
                mv, sem_in.at[b]),
            pltpu.make_async_copy(
                sig_hbm.at[pl.ds(pl.multiple_of(base * 4, CHUNK * 4),
                                 CHUNK * 4)],
                sv, sem_in.at[b]),
            pltpu.make_async_copy(
                eps_hbm.at[pl.ds(base, CHUNK)], ev, sem_in.at[b]),
        ]

    def out_copies(it, b):
        base = base_of(it)
        ov_p, ov_n = bufs[b][4], bufs[b][5]
        return [
            pltpu.make_async_copy(
                ov_p, out_hbm.at[pl.ds(base, CHUNK)], sem_out.at[b]),
            pltpu.make_async_copy(
                ov_n, out_hbm.at[pl.ds(N + base, CHUNK)], sem_out.at[b]),
            pltpu.make_async_copy(
                zero_v, outz_hbm.at[pl.ds(base, CHUNK)], sem_out.at[b]),
            pltpu.make_async_copy(
                zero_v, outz_hbm.at[pl.ds(N + base, CHUNK)], sem_out.at[b]),
        ]

    def compute(b):
        iv, mv, sv, ev, ov_p, ov_n = bufs[b]

        def _t_body(t, _):
            row = t >> 3
            d0 = (t & 7) * 16
            x = iv[pl.ds(t * 16, 16)]
            xp = jnp.maximum(x, 0.0).astype(jnp.int32)
            xn = jnp.maximum(-x, 0.0).astype(jnp.int32)
            cell0 = row * 1024 + d0
            vec = (row * 4096 + d0) + iota
            for k in range(8):
                sh = 7 - 2 * k
                if sh >= 0:
                    cp = (xp >> sh) & 0x180
                    cn = (xn >> sh) & 0x180
                else:
                    cp = (xp << -sh) & 0x180
                    cn = (xn << -sh) & 0x180
                veck = vec + k * 512
                mp = plsc.load_gather(mv, [veck + cp])
                sp = plsc.load_gather(sv, [veck + cp])
                mn = plsc.load_gather(mv, [veck + cn])
                sn = plsc.load_gather(sv, [veck + cn])
                off = cell0 + k * 128
                e = ev[pl.ds(off, 16)]
                ov_p[pl.ds(off, 16)] = mp + e * sp
                ov_n[pl.ds(off, 16)] = mn + e * sn
            return 0

        lax.fori_loop(0, T_PER_CHUNK, _t_body, 0)

    zvec = jnp.zeros((16,), jnp.float32)

    def zinit(g, _):
        zero_v[pl.ds(g * 16, 16)] = zvec
        return 0

    lax.fori_loop(0, CHUNK // 16, zinit, 0)

    for b in range(NBUF):
        for c in in_copies(b, b):
            c.start()

    def loop_ring(i, _):
        for b in range(NBUF):
            it = i * NBUF + b
            for c in in_copies(it, b):
                c.wait()

            @pl.when(it >= NBUF)
            def _():
                for c in out_copies(it - NBUF, b):
                    c.wait()

            compute(b)
            for c in out_copies(it, b):
                c.start()

            @pl.when(it + NBUF < ITERS)
            def _():
                for c in in_copies(it + NBUF, b):
                    c.start()
        return 0

    lax.fori_loop(0, ITERS // NBUF, loop_ring, 0)

    for b in range(NBUF):
        for c in out_copies(ITERS - NBUF + b, b):
            c.wait()


@jax.jit
def kernel(input, mean_G, sig_G, eps):
    mesh = plsc.VectorSubcoreMesh(core_axis_name="c", subcore_axis_name="s",
                                  num_cores=2, num_subcores=16)
    run = functools.partial(
        pl.kernel,
        out_type=(jax.ShapeDtypeStruct((2 * N,), jnp.float32),
                  jax.ShapeDtypeStruct((2 * N,), jnp.float32)),
        mesh=mesh,
        compiler_params=pltpu.CompilerParams(needs_layout_passes=False),
        scratch_types=(
            [pltpu.VMEM((CHUNK // 8,), jnp.float32),
             pltpu.VMEM((CHUNK * 4,), jnp.float32),
             pltpu.VMEM((CHUNK * 4,), jnp.float32),
             pltpu.VMEM((CHUNK,), jnp.float32),
             pltpu.VMEM((CHUNK,), jnp.float32),
             pltpu.VMEM((CHUNK,), jnp.float32)] * NBUF
            + [pltpu.VMEM((CHUNK,), jnp.float32),
               pltpu.SemaphoreType.DMA((NBUF,)),
               pltpu.SemaphoreType.DMA((NBUF,))]),
    )(_body)
    mean_lin = jnp.transpose(mean_G, (0, 1, 2, 4, 3)).reshape(-1)
    sig_lin = jnp.transpose(sig_G, (0, 1, 2, 4, 3)).reshape(-1)
    out, outz = run(input.reshape(-1), mean_lin, sig_lin, eps.reshape(-1))
    g_inactive = out.reshape(2, A, B, C0 * 8, D)
    g_active = outz.reshape(2, A, B, C0 * 8, D)
    return (g_active, g_inactive)
